# baseline (speedup 1.0000x reference)
import functools

import numpy as np
import jax
import jax.numpy as jnp
from jax.experimental import pallas as pl
from jax.experimental.pallas import tpu as pltpu

D = 1024
ROWS = 1024


def kernel(position_ids, table):
    batch, seq = position_ids.shape
    total = batch * seq
    n_blocks = total // ROWS

    j = np.arange(D)
    w = np.power(10000.0, -2.0 * np.floor(j / 2.0) / D)
    b = np.where(j % 2 == 0, 0.0, np.pi / 2.0)
    w32 = jnp.asarray(w, dtype=jnp.float32).reshape(1, D)
    b32 = jnp.asarray(b, dtype=jnp.float32).reshape(1, D)
    idx_col = position_ids.reshape(total, 1)

    def body(idx_ref, w_ref, b_ref, out_ref):
        p = idx_ref[...].astype(jnp.float32)
        ang = p * w_ref[...] + b_ref[...]
        out_ref[...] = jnp.sin(ang)

    out = pl.pallas_call(
        body,
        grid=(n_blocks,),
        in_specs=[
            pl.BlockSpec((ROWS, 1), lambda i: (i, 0)),
            pl.BlockSpec((1, D), lambda i: (0, 0)),
            pl.BlockSpec((1, D), lambda i: (0, 0)),
        ],
        out_specs=pl.BlockSpec((ROWS, D), lambda i: (i, 0)),
        out_shape=jax.ShapeDtypeStruct((total, D), jnp.float32),
        compiler_params=pltpu.CompilerParams(
            dimension_semantics=("parallel",),
        ),
    )(idx_col, w32, b32)
    return out.reshape(batch, seq, D)


# speedup vs baseline: 2.6624x; 2.6624x over previous
import functools

import numpy as np
import jax
import jax.numpy as jnp
from jax import lax
from jax.experimental import pallas as pl
from jax.experimental.pallas import tpu as pltpu

D = 1024
ROWS = 512
NHI = 128
NLO = 64


def _tables():
    j = np.arange(D)
    w = np.power(10000.0, -2.0 * np.floor(j / 2.0) / D)
    phi = np.where(j % 2 == 0, 0.0, np.pi / 2.0)
    h = np.arange(NHI)[:, None]
    l = np.arange(NLO)[:, None]
    a = h * NLO * w[None, :] + phi[None, :]
    b = l * w[None, :]
    wa = np.concatenate([np.sin(a), np.cos(a)], axis=1)
    wb = np.concatenate([np.cos(b), np.sin(b)], axis=1)

    def split(m):
        m32 = m.astype(np.float32)
        hi = m32.astype(jnp.bfloat16)
        lo = (m32 - hi.astype(np.float32)).astype(jnp.bfloat16)
        return jnp.asarray(hi), jnp.asarray(lo)

    return split(wa) + split(wb)


def kernel(position_ids, table):
    batch, seq = position_ids.shape
    total = batch * seq
    n_blocks = total // ROWS
    wa_hi, wa_lo, wb_hi, wb_lo = _tables()
    idx_col = position_ids.reshape(total, 1)

    def body(idx_ref, wah_ref, wal_ref, wbh_ref, wbl_ref, out_ref):
        p = idx_ref[...]
        hi = p >> 6
        lo = p & 63
        ih = lax.broadcasted_iota(jnp.int32, (ROWS, NHI), 1)
        il = lax.broadcasted_iota(jnp.int32, (ROWS, NLO), 1)
        oh_hi = (hi == ih).astype(jnp.bfloat16)
        oh_lo = (lo == il).astype(jnp.bfloat16)
        ga = (jnp.dot(oh_hi, wah_ref[...], preferred_element_type=jnp.float32)
              + jnp.dot(oh_hi, wal_ref[...], preferred_element_type=jnp.float32))
        gb = (jnp.dot(oh_lo, wbh_ref[...], preferred_element_type=jnp.float32)
              + jnp.dot(oh_lo, wbl_ref[...], preferred_element_type=jnp.float32))
        out_ref[...] = (ga[:, :D] * gb[:, :D] + ga[:, D:] * gb[:, D:])

    out = pl.pallas_call(
        body,
        grid=(n_blocks,),
        in_specs=[
            pl.BlockSpec((ROWS, 1), lambda i: (i, 0)),
            pl.BlockSpec((NHI, 2 * D), lambda i: (0, 0)),
            pl.BlockSpec((NHI, 2 * D), lambda i: (0, 0)),
            pl.BlockSpec((NLO, 2 * D), lambda i: (0, 0)),
            pl.BlockSpec((NLO, 2 * D), lambda i: (0, 0)),
        ],
        out_specs=pl.BlockSpec((ROWS, D), lambda i: (i, 0)),
        out_shape=jax.ShapeDtypeStruct((total, D), jnp.float32),
        compiler_params=pltpu.CompilerParams(
            dimension_semantics=("parallel",),
        ),
    )(idx_col, wa_hi, wa_lo, wb_hi, wb_lo)
    return out.reshape(batch, seq, D)


# speedup vs baseline: 4.4949x; 1.6883x over previous
import functools

import numpy as np
import jax
import jax.numpy as jnp
from jax import lax
from jax.experimental import pallas as pl
from jax.experimental.pallas import tpu as pltpu

D = 1024
ROWS = 512
NHI = 128
NLO = 64


def _tables():
    j = np.arange(D)
    w = np.power(10000.0, -2.0 * np.floor(j / 2.0) / D)
    phi = np.where(j % 2 == 0, 0.0, np.pi / 2.0)
    h = np.arange(NHI)[:, None]
    l = np.arange(NLO)[:, None]
    a = h * NLO * w[None, :] + phi[None, :]
    b = l * w[None, :]
    wa = np.concatenate([np.sin(a), np.cos(a)], axis=1)
    wb = np.concatenate([np.cos(b), np.sin(b)], axis=1)

    def split(m):
        m32 = m.astype(np.float32)
        hi = m32.astype(jnp.bfloat16)
        lo = (m32 - hi.astype(np.float32)).astype(jnp.bfloat16)
        return jnp.asarray(hi), jnp.asarray(lo)

    return split(wa) + split(wb)


def _tables_bf16():
    j = np.arange(D)
    w = np.power(10000.0, -2.0 * np.floor(j / 2.0) / D)
    phi = np.where(j % 2 == 0, 0.0, np.pi / 2.0)
    h = np.arange(NHI)[:, None]
    l = np.arange(NLO)[:, None]
    a = h * NLO * w[None, :] + phi[None, :]
    b = l * w[None, :]
    wa = np.concatenate([np.sin(a), np.cos(a)], axis=1)
    wb = np.concatenate([np.cos(b), np.sin(b)], axis=1)
    to16 = lambda m: jnp.asarray(m.astype(np.float32), dtype=jnp.bfloat16)
    return to16(wa), to16(wb)


def kernel(position_ids, table):
    batch, seq = position_ids.shape
    total = batch * seq
    n_blocks = total // ROWS
    wa, wb = _tables_bf16()
    idx_col = position_ids.reshape(total, 1)

    def body(idx_ref, wa_ref, wb_ref, out_ref):
        p = idx_ref[...]
        hi = p >> 6
        lo = p & 63
        ih = lax.broadcasted_iota(jnp.int32, (ROWS, NHI), 1)
        il = lax.broadcasted_iota(jnp.int32, (ROWS, NLO), 1)
        oh_hi = (hi == ih).astype(jnp.bfloat16)
        oh_lo = (lo == il).astype(jnp.bfloat16)
        ga = jnp.dot(oh_hi, wa_ref[...], preferred_element_type=jnp.float32)
        gb = jnp.dot(oh_lo, wb_ref[...], preferred_element_type=jnp.float32)
        out_ref[...] = (ga[:, :D] * gb[:, :D] + ga[:, D:] * gb[:, D:])

    out = pl.pallas_call(
        body,
        grid=(n_blocks,),
        in_specs=[
            pl.BlockSpec((ROWS, 1), lambda i: (i, 0)),
            pl.BlockSpec((NHI, 2 * D), lambda i: (0, 0)),
            pl.BlockSpec((NLO, 2 * D), lambda i: (0, 0)),
        ],
        out_specs=pl.BlockSpec((ROWS, D), lambda i: (i, 0)),
        out_shape=jax.ShapeDtypeStruct((total, D), jnp.float32),
        compiler_params=pltpu.CompilerParams(
            dimension_semantics=("parallel",),
        ),
    )(idx_col, wa, wb)
    return out.reshape(batch, seq, D)
